# baseline (speedup 1.0000x reference)
import functools

import jax
import jax.numpy as jnp
from jax import lax
from jax.experimental import pallas as pl
from jax.experimental.pallas import tpu as pltpu
from jax.experimental.pallas import tpu_sc as plsc

N = 10000
E = 320000
D = 128
NP = 10240
RPT = NP // 16
NW = 32
EPW = E // NW
K = 80
NCHUNK = EPW // K


def _mm_body(x_ref, w_ref, a2_ref, h_ref, s_ref):
    h = jnp.dot(x_ref[...], w_ref[...], preferred_element_type=jnp.float32)
    h_ref[...] = h
    s_ref[...] = lax.dot_general(
        a2_ref[...], h, (((0,), (1,)), ((), ())),
        preferred_element_type=jnp.float32,
    )


def _phase_a(x, W, A2):
    BN = 512
    grid = NP // BN
    return pl.pallas_call(
        _mm_body,
        grid=(grid,),
        in_specs=[
            pl.BlockSpec((BN, D), lambda i: (i, 0)),
            pl.BlockSpec((D, D), lambda i: (0, 0)),
            pl.BlockSpec((D, D), lambda i: (0, 0)),
        ],
        out_specs=[
            pl.BlockSpec((BN, D), lambda i: (i, 0)),
            pl.BlockSpec((D, BN), lambda i: (0, i)),
        ],
        out_shape=[
            jax.ShapeDtypeStruct((N, D), jnp.float32),
            jax.ShapeDtypeStruct((D, NP), jnp.float32),
        ],
    )(x, W, A2)


_MESH = plsc.VectorSubcoreMesh(
    core_axis_name="c", subcore_axis_name="s", num_cores=2, num_subcores=16
)


@functools.partial(
    pl.kernel,
    out_type=[
        jax.ShapeDtypeStruct((NP, D), jnp.float32),
        jax.ShapeDtypeStruct((NP, D), jnp.float32),
        jax.ShapeDtypeStruct((NP,), jnp.float32),
        jax.ShapeDtypeStruct((NP,), jnp.float32),
    ],
    mesh=_MESH,
    compiler_params=pltpu.CompilerParams(
        needs_layout_passes=False, use_tc_tiling_on_sc=False
    ),
    scratch_types=[
        pltpu.VMEM((NP,), jnp.float32),
        pltpu.VMEM((NP,), jnp.float32),
        [pltpu.VMEM((K,), jnp.int32)] * 2,
        [pltpu.VMEM((K,), jnp.int32)] * 2,
        [pltpu.VMEM((K,), jnp.int32)] * 2,
        [pltpu.VMEM((K,), jnp.float32)] * 2,
        [pltpu.VMEM((K, D), jnp.float32)] * 2,
        pltpu.VMEM_SHARED((NP, D), jnp.float32),
        pltpu.VMEM_SHARED((NP,), jnp.float32),
        pltpu.SemaphoreType.DMA,
        pltpu.SemaphoreType.DMA,
        pltpu.SemaphoreType.DMA,
    ],
)
def _phase_b(src_hbm, dst_hbm, s12t_hbm, h_hbm,
             u0, u1, den0, den1,
             s1_v, s2_v, srcb, dstb, scb, pbuf, rows, acc, dacc,
             gsem, isem, ssem):
    cid = lax.axis_index("c")
    sid = lax.axis_index("s")
    wid = sid * 2 + cid
    r0 = sid * RPT
    base0 = wid * EPW
    zf = jnp.zeros((16,), jnp.float32)

    pltpu.sync_copy(s12t_hbm.at[0], s1_v)
    pltpu.sync_copy(s12t_hbm.at[1], s2_v)

    def _zrow(i, _):
        for j in range(D // 16):
            rows[0][i, pl.ds(j * 16, 16)] = zf
        return 0
    lax.fori_loop(0, K, _zrow, 0)
    for i in range(K // 16):
        pbuf[0][pl.ds(i * 16, 16)] = zf
    for c in range(RPT // K):
        pltpu.async_copy(rows[0], acc.at[pl.ds(r0 + c * K, K)], ssem)
        pltpu.async_copy(pbuf[0], dacc.at[pl.ds(r0 + c * K, K)], ssem)
    for c in range(RPT // K):
        pltpu.make_async_copy(rows[0], acc.at[pl.ds(r0 + c * K, K)], ssem).wait()
        pltpu.make_async_copy(pbuf[0], dacc.at[pl.ds(r0 + c * K, K)], ssem).wait()
    plsc.subcore_barrier()

    def _idx_copy_start(t, b):
        base = base0 + t * K
        pltpu.async_copy(src_hbm.at[pl.ds(base, K)], srcb[b], isem)
        pltpu.async_copy(dst_hbm.at[pl.ds(base, K)], dstb[b], isem)

    def _idx_wait(b):
        pltpu.make_async_copy(src_hbm.at[pl.ds(0, K)], srcb[b], isem).wait()
        pltpu.make_async_copy(dst_hbm.at[pl.ds(0, K)], dstb[b], isem).wait()

    def _scatter_wait(b):
        pltpu.make_async_copy(rows[b], acc.at[scb[b]], ssem).wait()
        pltpu.make_async_copy(pbuf[b], dacc.at[scb[b]], ssem).wait()

    def _p_loop(b):
        for i in range(K // 16):
            isrc = srcb[b][pl.ds(i * 16, 16)]
            idst = dstb[b][pl.ds(i * 16, 16)]
            scb[b][pl.ds(i * 16, 16)] = isrc
            v = plsc.load_gather(s1_v, [isrc]) + plsc.load_gather(s2_v, [idst])
            e = jnp.where(v > 0, v, 0.2 * v)
            pbuf[b][pl.ds(i * 16, 16)] = jnp.exp(e)

    def _scale(b):
        def body(i2, _):
            for u in range(2):
                i = i2 * 2 + u
                bc = plsc.load_gather(pbuf[b], [lax.broadcast(i, (16,))])
                for j in range(D // 16):
                    rows[b][i, pl.ds(j * 16, 16)] = (
                        rows[b][i, pl.ds(j * 16, 16)] * bc
                    )
            return 0
        lax.fori_loop(0, K // 2, body, 0)

    def _scatter_start(b):
        pltpu.async_copy(rows[b], acc.at[scb[b]], ssem, add=True)
        pltpu.async_copy(pbuf[b], dacc.at[scb[b]], ssem, add=True)

    base = base0
    pltpu.sync_copy(src_hbm.at[pl.ds(base, K)], srcb[0])
    pltpu.sync_copy(dst_hbm.at[pl.ds(base, K)], dstb[0])
    pltpu.async_copy(h_hbm.at[dstb[0]], rows[0], gsem)
    _idx_copy_start(1, 1)

    def _iter(t, b, u):
        _p_loop(b)
        pltpu.make_async_copy(h_hbm.at[dstb[b]], rows[b], gsem).wait()

        @pl.when(t >= 1)
        def _():
            _scatter_wait(1 - b)
        _idx_wait(1 - b)
        pltpu.async_copy(h_hbm.at[dstb[1 - b]], rows[1 - b], gsem)

        @pl.when(t + 2 < NCHUNK)
        def _():
            _idx_copy_start(t + 2, b)

        _scatter_start(b)

    def _outer(u, _):
        _iter(2 * u, 0, u)
        _iter(2 * u + 1, 1, u)
        return 0
    lax.fori_loop(0, NCHUNK // 2, _outer, 0)

    _p_loop(0)
    pltpu.make_async_copy(h_hbm.at[dstb[0]], rows[0], gsem).wait()
    _scatter_start(0)
    _scatter_wait(1)
    _scatter_wait(0)
    plsc.subcore_barrier()

    @pl.when(cid == 0)
    def _():
        pltpu.sync_copy(acc.at[pl.ds(r0, RPT)], u0.at[pl.ds(r0, RPT)])
        pltpu.sync_copy(dacc.at[pl.ds(r0, RPT)], den0.at[pl.ds(r0, RPT)])

    @pl.when(cid == 1)
    def _():
        pltpu.sync_copy(acc.at[pl.ds(r0, RPT)], u1.at[pl.ds(r0, RPT)])
        pltpu.sync_copy(dacc.at[pl.ds(r0, RPT)], den1.at[pl.ds(r0, RPT)])


def _fin_body(u0_ref, u1_ref, d0_ref, d1_ref, o_ref):
    u = u0_ref[...] + u1_ref[...]
    d = d0_ref[...] + d1_ref[...]
    r = jnp.where(d > 0, 1.0 / jnp.where(d > 0, d, 1.0), 0.0)
    hp = u * r[:, None]
    o_ref[...] = jnp.where(hp > 0, hp, jnp.exp(jnp.minimum(hp, 0.0)) - 1.0)


def _phase_c(u0, u1, d0, d1):
    BN = 512
    grid = NP // BN
    return pl.pallas_call(
        _fin_body,
        grid=(grid,),
        in_specs=[
            pl.BlockSpec((BN, D), lambda i: (i, 0)),
            pl.BlockSpec((BN, D), lambda i: (i, 0)),
            pl.BlockSpec((BN,), lambda i: (i,)),
            pl.BlockSpec((BN,), lambda i: (i,)),
        ],
        out_specs=pl.BlockSpec((BN, D), lambda i: (i, 0)),
        out_shape=jax.ShapeDtypeStruct((N, D), jnp.float32),
    )(u0, u1, d0, d1)


def kernel(input, edge_list, W, a):
    A2 = jnp.zeros((D, D), jnp.float32)
    A2 = A2.at[:, 0].set(a[:D, 0]).at[:, 1].set(a[D:, 0])

    h, s12t = _phase_a(input, W, A2)

    src = edge_list[0]
    dst = edge_list[1]
    u0, u1, d0, d1 = _phase_b(src, dst, s12t, h)

    return _phase_c(u0, u1, d0, d1)
